# Initial kernel scaffold; baseline (speedup 1.0000x reference)
import jax
import jax.numpy as jnp
from jax.experimental import pallas as pl


def kernel(feat_idx, offsets, per_sample_weights, edge_index, w_ppi, w_self, emb_table, input_bias, W1, b1, W2, b2, Wout, bout):
    raise NotImplementedError("write your pallas kernel here")


# speedup vs baseline: 3.8254x; 3.8254x over previous
import functools

import jax
import jax.numpy as jnp
from jax import lax
from jax.experimental import pallas as pl
from jax.experimental.pallas import tpu as pltpu
from jax.experimental.pallas import tpu_sc as plsc

N = 10000
E = 320000
H = 128
LANES = 16
NC = 2
NS = 16
NW = NC * NS

EMB_CHUNK = 80
EMB_NCHUNK = N // EMB_CHUNK
EDGE_CHUNK = 128
EDGE_NCHUNK = E // EDGE_CHUNK
ROWS_PER_TILE = 624
ROWS_REMAIN = N - NS * ROWS_PER_TILE

_mesh = plsc.VectorSubcoreMesh(core_axis_name="c", subcore_axis_name="s")


_GDN = lax.GatherDimensionNumbers(
    offset_dims=(), collapsed_slice_dims=(0,), start_index_map=(0,))


def _lane_bcast(v16, lane):
    idx = jnp.full((LANES, 1), lane, jnp.int32)
    return lax.gather(v16, idx, _GDN, (1,),
                      mode=lax.GatherScatterMode.PROMISE_IN_BOUNDS)


def _scale_rows_inplace(rows_v, w_v, nrows):
    def body(g, carry):
        v16 = w_v[pl.ds(pl.multiple_of(g * LANES, LANES), LANES)]
        for lane in range(LANES):
            w = _lane_bcast(v16, lane)
            r = g * LANES + lane
            for c in range(H // LANES):
                sl = pl.ds(c * LANES, LANES)
                rows_v[r, sl] = rows_v[r, sl] * w
        return carry
    lax.fori_loop(0, nrows // LANES, body, 0)


@functools.partial(
    pl.kernel,
    mesh=_mesh,
    out_type=jax.ShapeDtypeStruct((N, H), jnp.float32),
    scratch_types=[
        pltpu.VMEM((EMB_CHUNK,), jnp.int32),
        pltpu.VMEM((EMB_CHUNK,), jnp.float32),
        pltpu.VMEM((EMB_CHUNK, H), jnp.float32),
        pltpu.VMEM((H,), jnp.float32),
        pltpu.SemaphoreType.DMA,
    ],
)
def _emb_call(feat_idx, psw, emb, bias, h0, idx_v, w_v, rows_v, bias_v, sem):
    wid = lax.axis_index("s") * NC + lax.axis_index("c")
    pltpu.sync_copy(bias, bias_v)
    nper = (EMB_NCHUNK + NW - 1) // NW
    for t in range(nper):
        j = wid + NW * t

        @pl.when(j < EMB_NCHUNK)
        def _():
            base = pl.multiple_of(j * EMB_CHUNK, EMB_CHUNK)
            pltpu.sync_copy(feat_idx.at[pl.ds(base, EMB_CHUNK)], idx_v)
            pltpu.sync_copy(psw.at[pl.ds(base, EMB_CHUNK)], w_v)
            pltpu.async_copy(emb.at[idx_v], rows_v, sem).wait()

            def body(g, carry):
                v16 = w_v[pl.ds(pl.multiple_of(g * LANES, LANES), LANES)]
                for lane in range(LANES):
                    w = _lane_bcast(v16, lane)
                    r = g * LANES + lane
                    for c in range(H // LANES):
                        sl = pl.ds(c * LANES, LANES)
                        rows_v[r, sl] = jnp.maximum(
                            rows_v[r, sl] * w + bias_v[sl], 0.0)
                return carry
            lax.fori_loop(0, EMB_CHUNK // LANES, body, 0)
            pltpu.sync_copy(rows_v, h0.at[pl.ds(base, EMB_CHUNK)])


@functools.partial(
    pl.kernel,
    mesh=_mesh,
    out_type=jax.ShapeDtypeStruct((2, N, H), jnp.float32),
    scratch_types=[
        pltpu.VMEM((EDGE_CHUNK,), jnp.int32),
        pltpu.VMEM((EDGE_CHUNK,), jnp.int32),
        pltpu.VMEM((EDGE_CHUNK,), jnp.float32),
        pltpu.VMEM((EDGE_CHUNK, H), jnp.float32),
        pltpu.VMEM_SHARED((N, H), jnp.float32),
        pltpu.SemaphoreType.DMA,
    ],
)
def _edge_call(h, src, dst, wboth, out2,
               sidx_v, didx_v, w_v, rows_v, acc, sem):
    cid = lax.axis_index("c")
    sid = lax.axis_index("s")

    def zbody(r, carry):
        for c in range(H // LANES):
            rows_v[r, pl.ds(c * LANES, LANES)] = jnp.zeros((LANES,), jnp.float32)
        return carry
    lax.fori_loop(0, EDGE_CHUNK, zbody, 0)
    row0 = sid * ROWS_PER_TILE
    for k in range(4):
        pltpu.sync_copy(rows_v, acc.at[pl.ds(row0 + k * EDGE_CHUNK, EDGE_CHUNK)])
    pltpu.sync_copy(rows_v.at[pl.ds(0, 112)], acc.at[pl.ds(row0 + 512, 112)])

    @pl.when(sid == NS - 1)
    def _():
        pltpu.sync_copy(rows_v.at[pl.ds(0, ROWS_REMAIN)],
                        acc.at[pl.ds(NS * ROWS_PER_TILE, ROWS_REMAIN)])
    plsc.subcore_barrier()

    nper = (EDGE_NCHUNK + NS - 1) // NS

    def ebody(i, carry):
        t = sid + NS * i

        @pl.when(t < EDGE_NCHUNK)
        def _():
            base = pl.multiple_of(t * EDGE_CHUNK, EDGE_CHUNK)
            pltpu.sync_copy(src.at[pl.ds(base, EDGE_CHUNK)], sidx_v)
            pltpu.sync_copy(dst.at[pl.ds(base, EDGE_CHUNK)], didx_v)
            wbase = pl.multiple_of(cid * E + t * EDGE_CHUNK, EDGE_CHUNK)
            pltpu.sync_copy(wboth.at[pl.ds(wbase, EDGE_CHUNK)], w_v)
            pltpu.async_copy(h.at[sidx_v], rows_v, sem).wait()
            _scale_rows_inplace(rows_v, w_v, EDGE_CHUNK)
            pltpu.sync_copy(rows_v, acc.at[didx_v], add=True)
        return carry
    lax.fori_loop(0, nper, ebody, 0)
    plsc.subcore_barrier()

    pltpu.sync_copy(acc.at[pl.ds(row0, ROWS_PER_TILE)],
                    out2.at[cid, pl.ds(row0, ROWS_PER_TILE)])

    @pl.when(sid == NS - 1)
    def _():
        pltpu.sync_copy(acc.at[pl.ds(NS * ROWS_PER_TILE, ROWS_REMAIN)],
                        out2.at[cid, pl.ds(NS * ROWS_PER_TILE, ROWS_REMAIN)])


BLK = 1000


def _layer_body(ppi_ref, res_ref, w_ref, b_ref, o_ref):
    z = lax.dot_general(ppi_ref[...], w_ref[...], (((1,), (1,)), ((), ())),
                        preferred_element_type=jnp.float32)
    o_ref[...] = jnp.maximum(z + b_ref[...], 0.0) + res_ref[...]


def _layer_update(ppi, res, W, b2d):
    return pl.pallas_call(
        _layer_body,
        grid=(N // BLK,),
        in_specs=[
            pl.BlockSpec((BLK, H), lambda i: (i, 0)),
            pl.BlockSpec((BLK, H), lambda i: (i, 0)),
            pl.BlockSpec((H, H), lambda i: (0, 0)),
            pl.BlockSpec((1, H), lambda i: (0, 0)),
        ],
        out_specs=pl.BlockSpec((BLK, H), lambda i: (i, 0)),
        out_shape=jax.ShapeDtypeStruct((N, H), jnp.float32),
    )(ppi, res, W, b2d)


def _final_body(ppi_ref, res_ref, w_ref, b_ref, wo_ref, bo_ref, o_ref):
    z = lax.dot_general(ppi_ref[...], w_ref[...], (((1,), (1,)), ((), ())),
                        preferred_element_type=jnp.float32)
    hcur = jnp.maximum(z + b_ref[...], 0.0) + res_ref[...]
    o_ref[...] = lax.dot_general(hcur, wo_ref[...], (((1,), (1,)), ((), ())),
                                 preferred_element_type=jnp.float32) + bo_ref[...]


def _final_update(ppi, res, W, b2d, wo_p, bo_p):
    return pl.pallas_call(
        _final_body,
        grid=(N // BLK,),
        in_specs=[
            pl.BlockSpec((BLK, H), lambda i: (i, 0)),
            pl.BlockSpec((BLK, H), lambda i: (i, 0)),
            pl.BlockSpec((H, H), lambda i: (0, 0)),
            pl.BlockSpec((1, H), lambda i: (0, 0)),
            pl.BlockSpec((H, H), lambda i: (0, 0)),
            pl.BlockSpec((1, H), lambda i: (0, 0)),
        ],
        out_specs=pl.BlockSpec((BLK, H), lambda i: (i, 0)),
        out_shape=jax.ShapeDtypeStruct((N, H), jnp.float32),
    )(ppi, res, W, b2d, wo_p, bo_p)


def kernel(feat_idx, offsets, per_sample_weights, edge_index, w_ppi, w_self,
           emb_table, input_bias, W1, b1, W2, b2, Wout, bout):
    del offsets
    src = edge_index[0].astype(jnp.int32)
    dst = edge_index[1].astype(jnp.int32)
    h0 = _emb_call(feat_idx.astype(jnp.int32), per_sample_weights,
                   emb_table, input_bias)
    wboth = jnp.concatenate([w_ppi, w_self])
    pair1 = _edge_call(h0, src, dst, wboth)
    h1 = _layer_update(pair1[0], pair1[1], W1, b1.reshape(1, H))
    pair2 = _edge_call(h1, src, dst, wboth)
    ppi2, res2 = pair2[0], pair2[1]
    C = Wout.shape[0]
    wo_p = jnp.zeros((H, H), jnp.float32).at[:C].set(Wout)
    bo_p = jnp.zeros((1, H), jnp.float32).at[0, :C].set(bout)
    out = _final_update(ppi2, res2, W2, b2.reshape(1, H), wo_p, bo_p)
    return out[:, :C]


# speedup vs baseline: 6.3230x; 1.6529x over previous
import functools

import jax
import jax.numpy as jnp
from jax import lax
from jax.experimental import pallas as pl
from jax.experimental.pallas import tpu as pltpu
from jax.experimental.pallas import tpu_sc as plsc

N = 10000
E = 320000
H = 128
LANES = 16
NC = 2
NS = 16
NW = NC * NS

EMB_CHUNK = 80
EMB_NCHUNK = N // EMB_CHUNK
EDGE_CHUNK = 128
EDGE_NCHUNK = E // EDGE_CHUNK
ROWS_PER_TILE = 624
ROWS_REMAIN = N - NS * ROWS_PER_TILE

_mesh = plsc.VectorSubcoreMesh(core_axis_name="c", subcore_axis_name="s")


_GDN = lax.GatherDimensionNumbers(
    offset_dims=(), collapsed_slice_dims=(0,), start_index_map=(0,))


def _lane_bcast(v16, lane):
    idx = jnp.full((LANES, 1), lane, jnp.int32)
    return lax.gather(v16, idx, _GDN, (1,),
                      mode=lax.GatherScatterMode.PROMISE_IN_BOUNDS)


def _scale_rows_inplace(rows_v, w_v, nrows):
    def body(g, carry):
        v16 = w_v[pl.ds(pl.multiple_of(g * LANES, LANES), LANES)]
        for lane in range(LANES):
            w = _lane_bcast(v16, lane)
            r = g * LANES + lane
            for c in range(H // LANES):
                sl = pl.ds(c * LANES, LANES)
                rows_v[r, sl] = rows_v[r, sl] * w
        return carry
    lax.fori_loop(0, nrows // LANES, body, 0)


@functools.partial(
    pl.kernel,
    mesh=_mesh,
    out_type=jax.ShapeDtypeStruct((N, H), jnp.float32),
    scratch_types=[
        pltpu.VMEM((EMB_CHUNK,), jnp.int32),
        pltpu.VMEM((EMB_CHUNK,), jnp.float32),
        pltpu.VMEM((EMB_CHUNK, H), jnp.float32),
        pltpu.VMEM((H,), jnp.float32),
        pltpu.SemaphoreType.DMA,
    ],
)
def _emb_call(feat_idx, psw, emb, bias, h0, idx_v, w_v, rows_v, bias_v, sem):
    wid = lax.axis_index("s") * NC + lax.axis_index("c")
    pltpu.sync_copy(bias, bias_v)
    nper = (EMB_NCHUNK + NW - 1) // NW
    for t in range(nper):
        j = wid + NW * t

        @pl.when(j < EMB_NCHUNK)
        def _():
            base = pl.multiple_of(j * EMB_CHUNK, EMB_CHUNK)
            pltpu.sync_copy(feat_idx.at[pl.ds(base, EMB_CHUNK)], idx_v)
            pltpu.sync_copy(psw.at[pl.ds(base, EMB_CHUNK)], w_v)
            pltpu.async_copy(emb.at[idx_v], rows_v, sem).wait()

            def body(g, carry):
                v16 = w_v[pl.ds(pl.multiple_of(g * LANES, LANES), LANES)]
                for lane in range(LANES):
                    w = _lane_bcast(v16, lane)
                    r = g * LANES + lane
                    for c in range(H // LANES):
                        sl = pl.ds(c * LANES, LANES)
                        rows_v[r, sl] = jnp.maximum(
                            rows_v[r, sl] * w + bias_v[sl], 0.0)
                return carry
            lax.fori_loop(0, EMB_CHUNK // LANES, body, 0)
            pltpu.sync_copy(rows_v, h0.at[pl.ds(base, EMB_CHUNK)])


@functools.partial(
    pl.kernel,
    mesh=_mesh,
    out_type=jax.ShapeDtypeStruct((2, N, H), jnp.float32),
    scratch_types=[
        pltpu.VMEM((2, EDGE_CHUNK), jnp.int32),
        pltpu.VMEM((2, EDGE_CHUNK), jnp.int32),
        pltpu.VMEM((EDGE_CHUNK,), jnp.float32),
        pltpu.VMEM((EDGE_CHUNK,), jnp.float32),
        pltpu.VMEM((EDGE_CHUNK, H), jnp.float32),
        pltpu.VMEM((EDGE_CHUNK, H), jnp.float32),
        pltpu.VMEM_SHARED((N, H), jnp.float32),
        pltpu.SemaphoreType.DMA,
        pltpu.SemaphoreType.DMA,
        pltpu.SemaphoreType.DMA,
        pltpu.SemaphoreType.DMA,
    ],
)
def _edge_call(h, eidx, wboth, out2,
               ed_a, ed_b, w_a, w_b, rows_a, rows_b, acc,
               sg_a, sg_b, ss_a, ss_b):
    cid = lax.axis_index("c")
    sid = lax.axis_index("s")
    ed = (ed_a, ed_b)
    w = (w_a, w_b)
    rows = (rows_a, rows_b)
    sg = (sg_a, sg_b)
    ss = (ss_a, ss_b)

    def zbody(r, carry):
        for c in range(H // LANES):
            rows_a[r, pl.ds(c * LANES, LANES)] = jnp.zeros((LANES,), jnp.float32)
        return carry
    lax.fori_loop(0, EDGE_CHUNK, zbody, 0)
    row0 = sid * ROWS_PER_TILE
    for k in range(4):
        pltpu.sync_copy(rows_a, acc.at[pl.ds(row0 + k * EDGE_CHUNK, EDGE_CHUNK)])
    pltpu.sync_copy(rows_a.at[pl.ds(0, 112)], acc.at[pl.ds(row0 + 512, 112)])

    @pl.when(sid == NS - 1)
    def _():
        pltpu.sync_copy(rows_a.at[pl.ds(0, ROWS_REMAIN)],
                        acc.at[pl.ds(NS * ROWS_PER_TILE, ROWS_REMAIN)])
    plsc.subcore_barrier()

    nper = (EDGE_NCHUNK + NS - 1) // NS

    def _wait_scatter(b):
        pltpu.make_async_copy(rows[b], acc.at[ed[b].at[1]], ss[b]).wait()

    def _phase_load(b, k):
        t = sid + NS * k

        @pl.when(t < EDGE_NCHUNK)
        def _():
            base = pl.multiple_of(t * EDGE_CHUNK, EDGE_CHUNK)
            pltpu.sync_copy(eidx.at[:, pl.ds(base, EDGE_CHUNK)], ed[b])
            wbase = pl.multiple_of(cid * E + t * EDGE_CHUNK, EDGE_CHUNK)
            pltpu.sync_copy(wboth.at[pl.ds(wbase, EDGE_CHUNK)], w[b])
            pltpu.async_copy(h.at[ed[b].at[0]], rows[b], sg[b])

    def _phase_compute(b, k):
        @pl.when((k >= 0) & (sid + NS * k < EDGE_NCHUNK))
        def _():
            pltpu.make_async_copy(h.at[ed[b].at[0]], rows[b], sg[b]).wait()
            _scale_rows_inplace(rows[b], w[b], EDGE_CHUNK)
            pltpu.async_copy(rows[b], acc.at[ed[b].at[1]], ss[b], add=True)

    def obody(o, carry):
        for b in range(2):
            k = 2 * o + b

            @pl.when((k >= 2) & (sid + NS * (k - 2) < EDGE_NCHUNK))
            def _():
                _wait_scatter(b)
            _phase_load(b, k)
            _phase_compute(1 - b, k - 1)
        return carry
    lax.fori_loop(0, (nper + 1) // 2, obody, 0)

    kl = nper - 1

    @pl.when(sid + NS * kl < EDGE_NCHUNK)
    def _():
        _wait_scatter(kl % 2)
    plsc.subcore_barrier()

    pltpu.sync_copy(acc.at[pl.ds(row0, ROWS_PER_TILE)],
                    out2.at[cid, pl.ds(row0, ROWS_PER_TILE)])

    @pl.when(sid == NS - 1)
    def _():
        pltpu.sync_copy(acc.at[pl.ds(NS * ROWS_PER_TILE, ROWS_REMAIN)],
                        out2.at[cid, pl.ds(NS * ROWS_PER_TILE, ROWS_REMAIN)])


BLK = 1000


def _layer_body(ppi_ref, res_ref, w_ref, b_ref, o_ref):
    z = lax.dot_general(ppi_ref[...], w_ref[...], (((1,), (1,)), ((), ())),
                        preferred_element_type=jnp.float32)
    o_ref[...] = jnp.maximum(z + b_ref[...], 0.0) + res_ref[...]


def _layer_update(ppi, res, W, b2d):
    return pl.pallas_call(
        _layer_body,
        grid=(N // BLK,),
        in_specs=[
            pl.BlockSpec((BLK, H), lambda i: (i, 0)),
            pl.BlockSpec((BLK, H), lambda i: (i, 0)),
            pl.BlockSpec((H, H), lambda i: (0, 0)),
            pl.BlockSpec((1, H), lambda i: (0, 0)),
        ],
        out_specs=pl.BlockSpec((BLK, H), lambda i: (i, 0)),
        out_shape=jax.ShapeDtypeStruct((N, H), jnp.float32),
    )(ppi, res, W, b2d)


def _final_body(ppi_ref, res_ref, w_ref, b_ref, wo_ref, bo_ref, o_ref):
    z = lax.dot_general(ppi_ref[...], w_ref[...], (((1,), (1,)), ((), ())),
                        preferred_element_type=jnp.float32)
    hcur = jnp.maximum(z + b_ref[...], 0.0) + res_ref[...]
    o_ref[...] = lax.dot_general(hcur, wo_ref[...], (((1,), (1,)), ((), ())),
                                 preferred_element_type=jnp.float32) + bo_ref[...]


def _final_update(ppi, res, W, b2d, wo_p, bo_p):
    return pl.pallas_call(
        _final_body,
        grid=(N // BLK,),
        in_specs=[
            pl.BlockSpec((BLK, H), lambda i: (i, 0)),
            pl.BlockSpec((BLK, H), lambda i: (i, 0)),
            pl.BlockSpec((H, H), lambda i: (0, 0)),
            pl.BlockSpec((1, H), lambda i: (0, 0)),
            pl.BlockSpec((H, H), lambda i: (0, 0)),
            pl.BlockSpec((1, H), lambda i: (0, 0)),
        ],
        out_specs=pl.BlockSpec((BLK, H), lambda i: (i, 0)),
        out_shape=jax.ShapeDtypeStruct((N, H), jnp.float32),
    )(ppi, res, W, b2d, wo_p, bo_p)


def kernel(feat_idx, offsets, per_sample_weights, edge_index, w_ppi, w_self,
           emb_table, input_bias, W1, b1, W2, b2, Wout, bout):
    del offsets
    eidx = edge_index.astype(jnp.int32)
    h0 = _emb_call(feat_idx.astype(jnp.int32), per_sample_weights,
                   emb_table, input_bias)
    wboth = jnp.concatenate([w_ppi, w_self])
    pair1 = _edge_call(h0, eidx, wboth)
    h1 = _layer_update(pair1[0], pair1[1], W1, b1.reshape(1, H))
    pair2 = _edge_call(h1, eidx, wboth)
    ppi2, res2 = pair2[0], pair2[1]
    C = Wout.shape[0]
    wo_p = jnp.zeros((H, H), jnp.float32).at[:C].set(Wout)
    bo_p = jnp.zeros((1, H), jnp.float32).at[0, :C].set(bout)
    out = _final_update(ppi2, res2, W2, b2.reshape(1, H), wo_p, bo_p)
    return out[:, :C]
